# baseline (speedup 1.0000x reference)
import jax
import jax.numpy as jnp
from jax.experimental import pallas as pl
from jax.experimental.pallas import tpu as pltpu

D_MODEL = 1024
E = 64
B = 64
GRP = 4


def _gating_kernel(x_ref, gw_ref, gb_ref, gs_out_ref, ids_out_ref):
    x = x_ref[...]
    gw = gw_ref[...]
    logits = jax.lax.dot_general(
        x, gw, (((1,), (1,)), ((), ())), preferred_element_type=jnp.float32
    ) + gb_ref[...]
    z = logits - jnp.max(logits, axis=1, keepdims=True)
    ez = jnp.exp(z)
    g = ez / jnp.sum(ez, axis=1, keepdims=True)
    m1 = jnp.max(g, axis=1, keepdims=True)
    g_wo_top1 = jnp.where(g == m1, -1.0, g)
    m2 = jnp.max(g_wo_top1, axis=1, keepdims=True)
    sel = (g >= m2).astype(jnp.float32)

    a_row = jnp.max(sel, axis=0, keepdims=True)
    io0 = jax.lax.broadcasted_iota(jnp.int32, (E, E), 0)
    io1 = jax.lax.broadcasted_iota(jnp.int32, (E, E), 1)
    upper = (io0 <= io1).astype(jnp.float32)
    cs_row = jnp.dot(a_row, upper, preferred_element_type=jnp.float32)
    pos_row = cs_row - 1.0
    n_act = jnp.sum(a_row)

    p_col = jax.lax.broadcasted_iota(jnp.int32, (E, 1), 0).astype(jnp.float32)
    q = (pos_row == p_col).astype(jnp.float32) * a_row
    gs_out_ref[...] = jax.lax.dot_general(
        g, q, (((1,), (1,)), ((), ())), preferred_element_type=jnp.float32
    )
    e_row = jax.lax.broadcasted_iota(jnp.int32, (1, E), 1).astype(jnp.float32)
    ids_col = jnp.sum(q * e_row, axis=1, keepdims=True)
    last_id = jnp.max(a_row * e_row - (1.0 - a_row))
    ids_filled = jnp.where(p_col < n_act, ids_col, last_id)
    ids_out_ref[...] = ids_filled.astype(jnp.int32)


def _expert_mm_kernel(ids_ref, x_ref, gs_ref, w0_ref, w1_ref, w2_ref, w3_ref, o_ref):
    i = pl.program_id(0)
    x = x_ref[...]
    iota0 = jax.lax.broadcasted_iota(jnp.int32, (E, GRP), 0)
    iota1 = jax.lax.broadcasted_iota(jnp.int32, (E, GRP), 1)
    onehot = (iota0 == GRP * i + iota1).astype(jnp.float32)
    cols = jnp.dot(gs_ref[...], onehot, preferred_element_type=jnp.float32)
    w_refs = (w0_ref, w1_ref, w2_ref, w3_ref)
    contrib = jnp.dot(x, w_refs[0][...], preferred_element_type=jnp.float32) * cols[:, 0:1]
    for k in range(1, GRP):
        contrib += (
            jnp.dot(x, w_refs[k][...], preferred_element_type=jnp.float32)
            * cols[:, k:k + 1]
        )
    o_ref[...] = jnp.where(i == 0, contrib, o_ref[...] + contrib)


def kernel(x, weight, gate_w, gate_b):
    gb2 = gate_b.reshape(1, E)

    gs, ids = pl.pallas_call(
        _gating_kernel,
        out_shape=(
            jax.ShapeDtypeStruct((B, E), jnp.float32),
            jax.ShapeDtypeStruct((E, 1), jnp.int32),
        ),
    )(x, gate_w, gb2)
    ids_flat = ids.reshape(E)

    def w_map(k):
        return lambda i, ids: (0, ids[GRP * i + k])

    grid_spec = pltpu.PrefetchScalarGridSpec(
        num_scalar_prefetch=1,
        grid=(E // GRP,),
        in_specs=[
            pl.BlockSpec((B, D_MODEL), lambda i, ids: (0, 0)),
            pl.BlockSpec((B, E), lambda i, ids: (0, 0)),
            pl.BlockSpec((D_MODEL, D_MODEL), w_map(0)),
            pl.BlockSpec((D_MODEL, D_MODEL), w_map(1)),
            pl.BlockSpec((D_MODEL, D_MODEL), w_map(2)),
            pl.BlockSpec((D_MODEL, D_MODEL), w_map(3)),
        ],
        out_specs=pl.BlockSpec((B, D_MODEL), lambda i, ids: (0, 0)),
    )
    out = pl.pallas_call(
        _expert_mm_kernel,
        grid_spec=grid_spec,
        out_shape=jax.ShapeDtypeStruct((B, D_MODEL), jnp.float32),
        compiler_params=pltpu.CompilerParams(
            dimension_semantics=("arbitrary",),
        ),
    )(ids_flat, x, gs, weight, weight, weight, weight)
    return out


# speedup vs baseline: 1.0386x; 1.0386x over previous
import jax
import jax.numpy as jnp
from jax.experimental import pallas as pl
from jax.experimental.pallas import tpu as pltpu

D_MODEL = 1024
E = 64
B = 64
GRP = 4


def _moe_kernel(x_ref, gw_ref, gb_ref, w_ref, o_ref, gs_ref):
    i = pl.program_id(0)

    @pl.when(i == 0)
    def _():
        x = x_ref[...]
        gw = gw_ref[...]
        logits = jax.lax.dot_general(
            x, gw, (((1,), (1,)), ((), ())), preferred_element_type=jnp.float32
        ) + gb_ref[...]
        z = logits - jnp.max(logits, axis=1, keepdims=True)
        ez = jnp.exp(z)
        g = ez / jnp.sum(ez, axis=1, keepdims=True)
        m1 = jnp.max(g, axis=1, keepdims=True)
        g_wo_top1 = jnp.where(g == m1, -1.0, g)
        m2 = jnp.max(g_wo_top1, axis=1, keepdims=True)
        sel = (g >= m2).astype(jnp.float32)
        active = jnp.max(sel, axis=0, keepdims=True)
        gs_ref[...] = g * active

    part = jnp.dot(x_ref[...], w_ref[...], preferred_element_type=jnp.float32)
    iota0 = jax.lax.broadcasted_iota(jnp.int32, (E, GRP), 0)
    iota1 = jax.lax.broadcasted_iota(jnp.int32, (E, GRP), 1)
    onehot = (iota0 == GRP * i + iota1).astype(jnp.float32)
    cols = jnp.dot(gs_ref[...], onehot, preferred_element_type=jnp.float32)
    contrib = part[:, :D_MODEL] * cols[:, 0:1]
    for k in range(1, GRP):
        contrib += part[:, k * D_MODEL:(k + 1) * D_MODEL] * cols[:, k:k + 1]
    o_ref[...] = jnp.where(i == 0, contrib, o_ref[...] + contrib)


def kernel(x, weight, gate_w, gate_b):
    gb2 = gate_b.reshape(1, E)
    out = pl.pallas_call(
        _moe_kernel,
        grid=(E // GRP,),
        in_specs=[
            pl.BlockSpec((B, D_MODEL), lambda i: (0, 0)),
            pl.BlockSpec((E, D_MODEL), lambda i: (0, 0)),
            pl.BlockSpec((1, E), lambda i: (0, 0)),
            pl.BlockSpec((D_MODEL, GRP * D_MODEL), lambda i: (0, i)),
        ],
        out_specs=pl.BlockSpec((B, D_MODEL), lambda i: (0, 0)),
        out_shape=jax.ShapeDtypeStruct((B, D_MODEL), jnp.float32),
        scratch_shapes=[pltpu.VMEM((B, E), jnp.float32)],
        compiler_params=pltpu.CompilerParams(
            dimension_semantics=("arbitrary",),
            vmem_limit_bytes=100 * 1024 * 1024,
        ),
    )(x, gate_w, gb2, weight)
    return out
